# Initial kernel scaffold; baseline (speedup 1.0000x reference)
import jax
import jax.numpy as jnp
from jax.experimental import pallas as pl


def kernel(x, edge_index, W1, b1, W2, b2, Wlin, blin):
    raise NotImplementedError("write your pallas kernel here")


# speedup vs baseline: 10.8513x; 10.8513x over previous
import functools

import jax
import jax.numpy as jnp
from jax import lax
from jax.experimental import pallas as pl
from jax.experimental.pallas import tpu as pltpu
from jax.experimental.pallas import tpu_sc as plsc

_N = 10000
_E = 320000
_NACC = 10240
_NW = 32
_CH = 128
_EPW = 10112
_EPAD = _NW * _EPW
_NCHUNK = _EPW // _CH
_RPT = _NACC // 16

_mesh = plsc.VectorSubcoreMesh(core_axis_name="c", subcore_axis_name="s")


_RED = _NACC // 10


@functools.partial(
    pl.kernel,
    out_type=jax.ShapeDtypeStruct((2 * _NACC,), jnp.float32),
    mesh=_mesh,
    scratch_types=[
        pltpu.VMEM((_CH,), jnp.int32),
        pltpu.VMEM((_NACC,), jnp.float32),
        pltpu.VMEM((_RED,), jnp.float32),
        pltpu.VMEM((_RED,), jnp.float32),
        pltpu.VMEM_SHARED((16, _NACC), jnp.float32),
    ],
    compiler_params=pltpu.CompilerParams(needs_layout_passes=False),
)
def _deg_kernel(dst_hbm, out_hbm, dst_v, degp_v, tmp_v, red_v, stage_sh):
    cid = lax.axis_index("c")
    sid = lax.axis_index("s")
    wid = sid * 2 + cid
    ones16 = jnp.full((16,), 1.0, jnp.float32)
    zeros16 = jnp.zeros((16,), jnp.float32)

    def _zero(t, carry):
        degp_v[pl.ds(t * 16, 16)] = zeros16
        return carry

    lax.fori_loop(0, _NACC // 16, _zero, 0)

    base = wid * _EPW

    def _body(i, carry):
        pltpu.sync_copy(dst_hbm.at[pl.ds(base + i * _CH, _CH)], dst_v)
        for j in range(_CH // 16):
            idx = dst_v[pl.ds(j * 16, 16)]
            plsc.addupdate_scatter(degp_v, [idx], ones16)
        return carry

    lax.fori_loop(0, _NCHUNK, _body, 0)

    pltpu.sync_copy(degp_v, stage_sh.at[sid])
    plsc.subcore_barrier()

    @pl.when(sid < _NACC // _RED)
    def _():
        def _zr(j, carry):
            red_v[pl.ds(j * 16, 16)] = zeros16
            return carry

        lax.fori_loop(0, _RED // 16, _zr, 0)
        for t in range(16):
            pltpu.sync_copy(stage_sh.at[t, pl.ds(sid * _RED, _RED)], tmp_v)

            def _acc(j, carry):
                s = pl.ds(j * 16, 16)
                red_v[s] = red_v[s] + tmp_v[s]
                return carry

            lax.fori_loop(0, _RED // 16, _acc, 0)
        pltpu.sync_copy(
            red_v, out_hbm.at[pl.ds(cid * _NACC + sid * _RED, _RED)])


@functools.partial(
    pl.kernel,
    out_type=jax.ShapeDtypeStruct((2, _NACC, 128), jnp.float32),
    mesh=_mesh,
    scratch_types=[
        pltpu.VMEM((_CH,), jnp.int32),
        pltpu.VMEM((_CH,), jnp.int32),
        pltpu.VMEM((_CH, 128), jnp.float32),
        pltpu.VMEM_SHARED((_NACC, 128), jnp.float32),
        pltpu.SemaphoreType.DMA,
    ],
)
def _scatter_kernel(g_hbm, src_hbm, dst_hbm, out_hbm,
                    src_v, dst_v, rows_v, acc_sh, sem):
    cid = lax.axis_index("c")
    sid = lax.axis_index("s")
    wid = sid * 2 + cid
    zeros16 = jnp.zeros((16,), jnp.float32)

    def _zero(t, carry):
        rows_v[t // 8, pl.ds((t % 8) * 16, 16)] = zeros16
        return carry

    lax.fori_loop(0, _CH * 8, _zero, 0)
    for k in range(_RPT // _CH):
        pltpu.sync_copy(rows_v, acc_sh.at[pl.ds(sid * _RPT + k * _CH, _CH)])
    plsc.subcore_barrier()

    base = wid * _EPW

    def _body(i, carry):
        pltpu.sync_copy(src_hbm.at[pl.ds(base + i * _CH, _CH)], src_v)
        pltpu.sync_copy(dst_hbm.at[pl.ds(base + i * _CH, _CH)], dst_v)
        pltpu.async_copy(g_hbm.at[src_v], rows_v, sem).wait()
        pltpu.sync_copy(rows_v, acc_sh.at[dst_v], add=True)
        return carry

    lax.fori_loop(0, _NCHUNK, _body, 0)
    plsc.subcore_barrier()

    for k in range(_RPT // _CH):
        r0 = sid * _RPT + k * _CH
        pltpu.sync_copy(acc_sh.at[pl.ds(r0, _CH)], rows_v)
        pltpu.sync_copy(rows_v, out_hbm.at[cid, pl.ds(r0, _CH)])


_BLK = 1000


def _dinv_from(deg_ref):
    d = deg_ref[...]
    return lax.rsqrt(d[:, 0:1] + d[:, 1:2] + 1.0)


def _tc1_body(x_ref, w1_ref, deg_ref, g1_ref):
    dinv = _dinv_from(deg_ref)
    h = jnp.dot(x_ref[...], w1_ref[...], preferred_element_type=jnp.float32)
    g1_ref[...] = h * dinv


def _tc2_body(p0_ref, p1_ref, g1_ref, deg_ref, b1_ref, w2_ref, g2_ref):
    dinv = _dinv_from(deg_ref)
    h1 = (p0_ref[0] + p1_ref[0] + g1_ref[...]) * dinv + b1_ref[...]
    h1 = jnp.maximum(h1, 0.0)
    h2 = jnp.dot(h1, w2_ref[...], preferred_element_type=jnp.float32)
    g2_ref[...] = jnp.concatenate([h2 * dinv, jnp.zeros_like(h2)], axis=1)


def _tc3_body(q0_ref, q1_ref, g2_ref, deg_ref, b2_ref, wl_ref, bl_ref,
              out_ref):
    dinv = _dinv_from(deg_ref)
    acc = q0_ref[0][:, 0:64] + q1_ref[0][:, 0:64] + g2_ref[:, 0:64]
    h2 = acc * dinv + b2_ref[...]
    f = jnp.dot(h2, wl_ref[...], preferred_element_type=jnp.float32)
    f = f + bl_ref[...]
    m = jnp.max(f, axis=1, keepdims=True)
    e = jnp.exp(f - m)
    out_ref[...] = (f - m) - jnp.log(jnp.sum(e, axis=1, keepdims=True))


def _row_spec(w):
    return pl.BlockSpec((_BLK, w), lambda i: (i, 0))


def _full_spec(shape):
    nd = len(shape)
    return pl.BlockSpec(shape, lambda i: (0,) * nd)


def _part_spec(j, w):
    return pl.BlockSpec((1, _BLK, w), lambda i, j=j: (j, i, 0))


_tc1 = pl.pallas_call(
    _tc1_body,
    grid=(_N // _BLK,),
    in_specs=[_row_spec(128), _full_spec((128, 128)), _row_spec(2)],
    out_specs=_row_spec(128),
    out_shape=jax.ShapeDtypeStruct((_N, 128), jnp.float32),
)

_tc2 = pl.pallas_call(
    _tc2_body,
    grid=(_N // _BLK,),
    in_specs=[_part_spec(0, 128), _part_spec(1, 128), _row_spec(128),
              _row_spec(2), _full_spec((1, 128)), _full_spec((128, 64))],
    out_specs=_row_spec(128),
    out_shape=jax.ShapeDtypeStruct((_N, 128), jnp.float32),
)

_tc3 = pl.pallas_call(
    _tc3_body,
    grid=(_N // _BLK,),
    in_specs=[_part_spec(0, 128), _part_spec(1, 128), _row_spec(128),
              _row_spec(2), _full_spec((1, 64)), _full_spec((64, 64)),
              _full_spec((1, 64))],
    out_specs=_row_spec(64),
    out_shape=jax.ShapeDtypeStruct((_N, 64), jnp.float32),
)


def kernel(x, edge_index, W1, b1, W2, b2, Wlin, blin):
    pad = _EPAD - _E
    src = jnp.concatenate([edge_index[0], jnp.zeros((pad,), jnp.int32)])
    dst = jnp.concatenate([edge_index[1], jnp.full((pad,), _N, jnp.int32)])

    deg_parts = _deg_kernel(dst)
    deg_t = deg_parts.reshape(2, _NACC).T

    g1 = _tc1(x, W1, deg_t)
    p = _scatter_kernel(g1, src, dst)
    g2 = _tc2(p, p, g1, deg_t, b1.reshape(1, 128), W2)
    q = _scatter_kernel(g2, src, dst)
    return _tc3(q, q, g2, deg_t, b2.reshape(1, 64), Wlin,
                blin.reshape(1, 64))
